# baseline (speedup 1.0000x reference)
import functools

import jax
import jax.numpy as jnp
from jax import lax
from jax.experimental import pallas as pl
from jax.experimental.pallas import tpu as pltpu
from jax.experimental.pallas import tpu_sc as plsc

N = 10000
E = 320000
DIN = 128
DH = 256
DOUT = 128

NC = 2
NS = 16
NW = NC * NS
CH = 96
STEPS = 108
HALF = STEPS // 2
N_TRASH = 8
N_ACC = N + N_TRASH

RPT = 624
R_TAIL_BASE = 16 * RPT
R_TAIL = N_ACC - 16 * RPT
W_TAIL = N - 16 * RPT
NP = 10240


def _row_ranges(sid):
    return sid * RPT


def _seg_sum_body(with_deg, *refs):
    if with_deg:
        (feat_hbm, src_hbm, dst_hbm, zf_hbm, ones_hbm,
         aggp_hbm, degp_hbm,
         src_v, dst_v, rows_a, rows_b, ones_v, dbuf, agg_sh, deg_sh,
         sem, sem2) = refs
    else:
        (feat_hbm, src_hbm, dst_hbm, zf_hbm,
         aggp_hbm,
         src_v, dst_v, rows_a, rows_b, agg_sh, sem, sem2) = refs

    cid = lax.axis_index("c")
    sid = lax.axis_index("s")
    wid = sid * NC + cid
    r0 = sid * RPT

    pltpu.sync_copy(zf_hbm.at[pl.ds(r0, RPT)], agg_sh.at[pl.ds(r0, RPT)])
    if with_deg:
        @pl.loop(0, RPT // 16)
        def _(i):
            dbuf[pl.ds(i * 16, 16)] = jnp.zeros((16,), jnp.float32)
        pltpu.sync_copy(dbuf, deg_sh.at[pl.ds(r0, RPT)])

    @pl.when(sid == NS - 1)
    def _():
        pltpu.sync_copy(zf_hbm.at[pl.ds(R_TAIL_BASE, R_TAIL)],
                        agg_sh.at[pl.ds(R_TAIL_BASE, R_TAIL)])
        if with_deg:
            pltpu.sync_copy(dbuf.at[pl.ds(0, R_TAIL)],
                            deg_sh.at[pl.ds(R_TAIL_BASE, R_TAIL)])

    if with_deg:
        pltpu.sync_copy(ones_hbm, ones_v)

    plsc.subcore_barrier()

    def issue(s, buf):
        return pltpu.async_copy(feat_hbm.at[src_v.at[s]], buf, sem)

    def scatter(s, buf):
        pltpu.sync_copy(buf, agg_sh.at[dst_v.at[s]], add=True)
        if with_deg:
            pltpu.sync_copy(ones_v, deg_sh.at[dst_v.at[s]], add=True)

    def scatter_async(s, buf):
        pltpu.async_copy(buf, agg_sh.at[dst_v.at[s]], sem2, add=True)
        if with_deg:
            pltpu.async_copy(ones_v, deg_sh.at[dst_v.at[s]], sem2, add=True)

    def scatter_drain(s, buf):
        pltpu.make_async_copy(buf, agg_sh.at[dst_v.at[s]], sem2).wait()
        if with_deg:
            pltpu.make_async_copy(ones_v, deg_sh.at[dst_v.at[s]], sem2).wait()

    for h in range(2):
        pltpu.sync_copy(src_hbm.at[2 * wid + h], src_v)
        pltpu.sync_copy(dst_hbm.at[2 * wid + h], dst_v)
        issue(0, rows_a)

        @pl.loop(0, HALF // 2)
        def _(t):
            s0 = 2 * t
            s1 = s0 + 1

            @pl.when(t > 0)
            def _():
                scatter_drain(s0, rows_b)
            pltpu.make_async_copy(feat_hbm.at[pl.ds(0, CH)],
                                  rows_a, sem).wait()
            db = issue(s1, rows_b)
            scatter_async(s0, rows_a)
            db.wait()
            scatter_drain(s0, rows_a)

            @pl.when(s1 + 1 < HALF)
            def _():
                issue(s1 + 1, rows_a)
            scatter_async(s1, rows_b)

        scatter_drain(HALF - 1, rows_b)

    plsc.subcore_barrier()

    pltpu.sync_copy(agg_sh.at[pl.ds(r0, RPT)],
                    aggp_hbm.at[pl.ds(cid * NP + r0, RPT)])
    if with_deg:
        pltpu.sync_copy(deg_sh.at[pl.ds(r0, RPT)], dbuf)
        pltpu.sync_copy(dbuf, degp_hbm.at[pl.ds(cid * NP + r0, RPT)])

    @pl.when(sid == NS - 1)
    def _():
        pltpu.sync_copy(agg_sh.at[pl.ds(R_TAIL_BASE, W_TAIL)],
                        aggp_hbm.at[pl.ds(cid * NP + R_TAIL_BASE, W_TAIL)])
        if with_deg:
            pltpu.sync_copy(deg_sh.at[pl.ds(R_TAIL_BASE, W_TAIL)],
                            dbuf.at[pl.ds(0, W_TAIL)])
            pltpu.sync_copy(dbuf.at[pl.ds(0, W_TAIL)],
                            degp_hbm.at[pl.ds(cid * NP + R_TAIL_BASE, W_TAIL)])


def _make_seg_sum(with_deg):
    mesh = plsc.VectorSubcoreMesh(core_axis_name="c", subcore_axis_name="s",
                                  num_cores=NC, num_subcores=NS)
    if with_deg:
        out_type = (
            jax.ShapeDtypeStruct((NC * NP, DIN), jnp.float32),
            jax.ShapeDtypeStruct((NC * NP,), jnp.float32),
        )
    else:
        out_type = jax.ShapeDtypeStruct((NC * NP, DIN), jnp.float32)
    scratch = [
        pltpu.VMEM((HALF, CH), jnp.int32),
        pltpu.VMEM((HALF, CH), jnp.int32),
        pltpu.VMEM((CH, DIN), jnp.float32),
        pltpu.VMEM((CH, DIN), jnp.float32),
    ]
    if with_deg:
        scratch.append(pltpu.VMEM((CH,), jnp.float32))
        scratch.append(pltpu.VMEM((RPT,), jnp.float32))
    scratch.append(pltpu.VMEM_SHARED((N_ACC, DIN), jnp.float32))
    if with_deg:
        scratch.append(pltpu.VMEM_SHARED((N_ACC,), jnp.float32))
    scratch.append(pltpu.SemaphoreType.DMA)
    scratch.append(pltpu.SemaphoreType.DMA)
    return pl.kernel(
        functools.partial(_seg_sum_body, with_deg),
        out_type=out_type,
        mesh=mesh,
        scratch_types=scratch,
    )


@functools.cache
def _get_seg_sum(with_deg):
    return _make_seg_sum(with_deg)


def _seg_sum_deg(*args):
    return _get_seg_sum(True)(*args)


def _seg_sum(*args):
    return _get_seg_sum(False)(*args)


RB = 1024
GRID = NP // RB


def _layer1_body(a0, a1, d0, d1, x, w1l, w1r, w2l, w2r, b1, b2, hl_o, hr_o):
    agg = a0[...] + a1[...]
    deg = jnp.maximum(d0[...] + d1[...], 1.0)
    mean = agg * (1.0 / deg)[:, None]
    h = mean @ w1l[...] + x[...] @ w1r[...] + b1[...]
    h = jnp.maximum(h, 0.0)
    hl_o[...] = h @ w2l[...]
    hr_o[...] = h @ w2r[...] + b2[...]


def _layer1(aggp, degp, x, w1l, w1r, w2l, w2r, b1, b2):
    return pl.pallas_call(
        _layer1_body,
        grid=(GRID,),
        in_specs=[
            pl.BlockSpec((RB, DIN), lambda i: (i, 0)),
            pl.BlockSpec((RB, DIN), lambda i: (GRID + i, 0)),
            pl.BlockSpec((RB,), lambda i: (i,)),
            pl.BlockSpec((RB,), lambda i: (GRID + i,)),
            pl.BlockSpec((RB, DIN), lambda i: (i, 0)),
            pl.BlockSpec((DIN, DH), lambda i: (0, 0)),
            pl.BlockSpec((DIN, DH), lambda i: (0, 0)),
            pl.BlockSpec((DH, DOUT), lambda i: (0, 0)),
            pl.BlockSpec((DH, DOUT), lambda i: (0, 0)),
            pl.BlockSpec((1, DH), lambda i: (0, 0)),
            pl.BlockSpec((1, DOUT), lambda i: (0, 0)),
        ],
        out_specs=[
            pl.BlockSpec((RB, DOUT), lambda i: (i, 0)),
            pl.BlockSpec((RB, DOUT), lambda i: (i, 0)),
        ],
        out_shape=[
            jax.ShapeDtypeStruct((N, DOUT), jnp.float32),
            jax.ShapeDtypeStruct((N, DOUT), jnp.float32),
        ],
    )(aggp, aggp, degp, degp, x, w1l, w1r, w2l, w2r, b1, b2)


def _final_body(a0, a1, d0, d1, hr, out_o, lsm_o):
    agg = a0[...] + a1[...]
    deg = jnp.maximum(d0[...] + d1[...], 1.0)
    o = agg * (1.0 / deg)[:, None] + hr[...]
    m = jnp.max(o, axis=1, keepdims=True)
    e = jnp.exp(o - m)
    s = jnp.sum(e, axis=1, keepdims=True)
    out_o[...] = o
    lsm_o[...] = (o - m) - jnp.log(s)


def _final(aggp2, degp, hr):
    return pl.pallas_call(
        _final_body,
        grid=(GRID,),
        in_specs=[
            pl.BlockSpec((RB, DOUT), lambda i: (i, 0)),
            pl.BlockSpec((RB, DOUT), lambda i: (GRID + i, 0)),
            pl.BlockSpec((RB,), lambda i: (i,)),
            pl.BlockSpec((RB,), lambda i: (GRID + i,)),
            pl.BlockSpec((RB, DOUT), lambda i: (i, 0)),
        ],
        out_specs=[
            pl.BlockSpec((RB, DOUT), lambda i: (i, 0)),
            pl.BlockSpec((RB, DOUT), lambda i: (i, 0)),
        ],
        out_shape=[
            jax.ShapeDtypeStruct((N, DOUT), jnp.float32),
            jax.ShapeDtypeStruct((N, DOUT), jnp.float32),
        ],
    )(aggp2, aggp2, degp, degp, hr)


def _pad_edges(edge_index, steps, ch):
    npad = NW * steps * ch - E
    pad_src = (jnp.arange(npad, dtype=jnp.int32) * 37) % N
    pad_dst = N + (jnp.arange(npad, dtype=jnp.int32) % N_TRASH)
    src = jnp.concatenate([edge_index[0], pad_src]).reshape(NW, steps, ch)
    dst = jnp.concatenate([edge_index[1], pad_dst]).reshape(NW, steps, ch)
    return src, dst


def kernel(x, edge_index, W1l, b1, W1r, W2l, b2, W2r):
    src2, dst2 = _pad_edges(edge_index, STEPS, CH)
    src2 = src2.reshape(NW * 2, HALF, CH)
    dst2 = dst2.reshape(NW * 2, HALF, CH)
    src1, dst1 = src2, dst2
    zf = jnp.zeros((N_ACC, DIN), jnp.float32)
    ones = jnp.ones((CH,), jnp.float32)

    aggp1, degp = _seg_sum_deg(x, src1, dst1, zf, ones)
    hl, hr = _layer1(aggp1, degp, x, W1l.T, W1r.T, W2l.T, W2r.T,
                     b1.reshape(1, DH), b2.reshape(1, DOUT))
    aggp2 = _seg_sum(hl, src2, dst2, zf)
    out, lsm = _final(aggp2, degp, hr)
    return (out, lsm)


# speedup vs baseline: 1.0058x; 1.0058x over previous
import functools

import jax
import jax.numpy as jnp
from jax import lax
from jax.experimental import pallas as pl
from jax.experimental.pallas import tpu as pltpu
from jax.experimental.pallas import tpu_sc as plsc

N = 10000
E = 320000
DIN = 128
DH = 256
DOUT = 128

NC = 2
NS = 16
NW = NC * NS
CH = 96
STEPS = 108
HALF = STEPS // 2
N_TRASH = 8
N_ACC = N + N_TRASH

RPT = 624
R_TAIL_BASE = 16 * RPT
R_TAIL = N_ACC - 16 * RPT
W_TAIL = N - 16 * RPT
NP = 10240


def _row_ranges(sid):
    return sid * RPT


def _seg_sum_body(with_deg, *refs):
    if with_deg:
        (feat_hbm, src_hbm, dst_hbm, zf_hbm, ones_hbm,
         aggp_hbm, degp_hbm,
         src_v, dst_v, rows_a, rows_b, ones_v, dbuf, agg_sh, deg_sh,
         sem) = refs
    else:
        (feat_hbm, src_hbm, dst_hbm, zf_hbm,
         aggp_hbm,
         src_v, dst_v, rows_a, rows_b, agg_sh, sem) = refs

    cid = lax.axis_index("c")
    sid = lax.axis_index("s")
    wid = sid * NC + cid
    r0 = sid * RPT

    pltpu.sync_copy(zf_hbm.at[pl.ds(r0, RPT)], agg_sh.at[pl.ds(r0, RPT)])
    if with_deg:
        @pl.loop(0, RPT // 16)
        def _(i):
            dbuf[pl.ds(i * 16, 16)] = jnp.zeros((16,), jnp.float32)
        pltpu.sync_copy(dbuf, deg_sh.at[pl.ds(r0, RPT)])

    @pl.when(sid == NS - 1)
    def _():
        pltpu.sync_copy(zf_hbm.at[pl.ds(R_TAIL_BASE, R_TAIL)],
                        agg_sh.at[pl.ds(R_TAIL_BASE, R_TAIL)])
        if with_deg:
            pltpu.sync_copy(dbuf.at[pl.ds(0, R_TAIL)],
                            deg_sh.at[pl.ds(R_TAIL_BASE, R_TAIL)])

    if with_deg:
        pltpu.sync_copy(ones_hbm, ones_v)

    plsc.subcore_barrier()

    def issue(s, buf):
        return pltpu.async_copy(feat_hbm.at[src_v.at[s]], buf, sem)

    def scatter(s, buf):
        pltpu.sync_copy(buf, agg_sh.at[dst_v.at[s]], add=True)
        if with_deg:
            pltpu.sync_copy(ones_v, deg_sh.at[dst_v.at[s]], add=True)

    for h in range(2):
        pltpu.sync_copy(src_hbm.at[2 * wid + h], src_v)
        pltpu.sync_copy(dst_hbm.at[2 * wid + h], dst_v)
        issue(0, rows_a)

        @pl.loop(0, HALF // 2)
        def _(t):
            s0 = 2 * t
            s1 = s0 + 1
            pltpu.make_async_copy(feat_hbm.at[pl.ds(0, CH)],
                                  rows_a, sem).wait()
            db = issue(s1, rows_b)
            scatter(s0, rows_a)
            db.wait()

            @pl.when(s1 + 1 < HALF)
            def _():
                issue(s1 + 1, rows_a)
            scatter(s1, rows_b)

    plsc.subcore_barrier()

    pltpu.sync_copy(agg_sh.at[pl.ds(r0, RPT)],
                    aggp_hbm.at[pl.ds(cid * NP + r0, RPT)])
    if with_deg:
        pltpu.sync_copy(deg_sh.at[pl.ds(r0, RPT)], dbuf)
        pltpu.sync_copy(dbuf, degp_hbm.at[pl.ds(cid * NP + r0, RPT)])

    @pl.when(sid == NS - 1)
    def _():
        pltpu.sync_copy(agg_sh.at[pl.ds(R_TAIL_BASE, W_TAIL)],
                        aggp_hbm.at[pl.ds(cid * NP + R_TAIL_BASE, W_TAIL)])
        if with_deg:
            pltpu.sync_copy(deg_sh.at[pl.ds(R_TAIL_BASE, W_TAIL)],
                            dbuf.at[pl.ds(0, W_TAIL)])
            pltpu.sync_copy(dbuf.at[pl.ds(0, W_TAIL)],
                            degp_hbm.at[pl.ds(cid * NP + R_TAIL_BASE, W_TAIL)])


def _make_seg_sum(with_deg):
    mesh = plsc.VectorSubcoreMesh(core_axis_name="c", subcore_axis_name="s",
                                  num_cores=NC, num_subcores=NS)
    if with_deg:
        out_type = (
            jax.ShapeDtypeStruct((NC * NP, DIN), jnp.float32),
            jax.ShapeDtypeStruct((NC * NP,), jnp.float32),
        )
    else:
        out_type = jax.ShapeDtypeStruct((NC * NP, DIN), jnp.float32)
    scratch = [
        pltpu.VMEM((HALF, CH), jnp.int32),
        pltpu.VMEM((HALF, CH), jnp.int32),
        pltpu.VMEM((CH, DIN), jnp.float32),
        pltpu.VMEM((CH, DIN), jnp.float32),
    ]
    if with_deg:
        scratch.append(pltpu.VMEM((CH,), jnp.float32))
        scratch.append(pltpu.VMEM((RPT,), jnp.float32))
    scratch.append(pltpu.VMEM_SHARED((N_ACC, DIN), jnp.float32))
    if with_deg:
        scratch.append(pltpu.VMEM_SHARED((N_ACC,), jnp.float32))
    scratch.append(pltpu.SemaphoreType.DMA)
    return pl.kernel(
        functools.partial(_seg_sum_body, with_deg),
        out_type=out_type,
        mesh=mesh,
        scratch_types=scratch,
    )


@functools.cache
def _get_seg_sum(with_deg):
    return _make_seg_sum(with_deg)


def _seg_sum_deg(*args):
    return _get_seg_sum(True)(*args)


def _seg_sum(*args):
    return _get_seg_sum(False)(*args)


RB = 1024
GRID = NP // RB


def _layer1_body(a0, a1, d0, d1, x, w1l, w1r, w2l, w2r, b1, b2, hl_o, hr_o):
    agg = a0[...] + a1[...]
    deg = jnp.maximum(d0[...] + d1[...], 1.0)
    mean = agg * (1.0 / deg)[:, None]
    h = mean @ w1l[...] + x[...] @ w1r[...] + b1[...]
    h = jnp.maximum(h, 0.0)
    hl_o[...] = h @ w2l[...]
    hr_o[...] = h @ w2r[...] + b2[...]


def _layer1(aggp, degp, x, w1l, w1r, w2l, w2r, b1, b2):
    return pl.pallas_call(
        _layer1_body,
        grid=(GRID,),
        in_specs=[
            pl.BlockSpec((RB, DIN), lambda i: (i, 0)),
            pl.BlockSpec((RB, DIN), lambda i: (GRID + i, 0)),
            pl.BlockSpec((RB,), lambda i: (i,)),
            pl.BlockSpec((RB,), lambda i: (GRID + i,)),
            pl.BlockSpec((RB, DIN), lambda i: (i, 0)),
            pl.BlockSpec((DIN, DH), lambda i: (0, 0)),
            pl.BlockSpec((DIN, DH), lambda i: (0, 0)),
            pl.BlockSpec((DH, DOUT), lambda i: (0, 0)),
            pl.BlockSpec((DH, DOUT), lambda i: (0, 0)),
            pl.BlockSpec((1, DH), lambda i: (0, 0)),
            pl.BlockSpec((1, DOUT), lambda i: (0, 0)),
        ],
        out_specs=[
            pl.BlockSpec((RB, DOUT), lambda i: (i, 0)),
            pl.BlockSpec((RB, DOUT), lambda i: (i, 0)),
        ],
        out_shape=[
            jax.ShapeDtypeStruct((N, DOUT), jnp.float32),
            jax.ShapeDtypeStruct((N, DOUT), jnp.float32),
        ],
    )(aggp, aggp, degp, degp, x, w1l, w1r, w2l, w2r, b1, b2)


def _final_body(a0, a1, d0, d1, hr, out_o, lsm_o):
    agg = a0[...] + a1[...]
    deg = jnp.maximum(d0[...] + d1[...], 1.0)
    o = agg * (1.0 / deg)[:, None] + hr[...]
    m = jnp.max(o, axis=1, keepdims=True)
    e = jnp.exp(o - m)
    s = jnp.sum(e, axis=1, keepdims=True)
    out_o[...] = o
    lsm_o[...] = (o - m) - jnp.log(s)


def _final(aggp2, degp, hr):
    return pl.pallas_call(
        _final_body,
        grid=(GRID,),
        in_specs=[
            pl.BlockSpec((RB, DOUT), lambda i: (i, 0)),
            pl.BlockSpec((RB, DOUT), lambda i: (GRID + i, 0)),
            pl.BlockSpec((RB,), lambda i: (i,)),
            pl.BlockSpec((RB,), lambda i: (GRID + i,)),
            pl.BlockSpec((RB, DOUT), lambda i: (i, 0)),
        ],
        out_specs=[
            pl.BlockSpec((RB, DOUT), lambda i: (i, 0)),
            pl.BlockSpec((RB, DOUT), lambda i: (i, 0)),
        ],
        out_shape=[
            jax.ShapeDtypeStruct((N, DOUT), jnp.float32),
            jax.ShapeDtypeStruct((N, DOUT), jnp.float32),
        ],
    )(aggp2, aggp2, degp, degp, hr)


def _pad_edges(edge_index, steps, ch):
    npad = NW * steps * ch - E
    pad_src = (jnp.arange(npad, dtype=jnp.int32) * 37) % N
    pad_dst = N + (jnp.arange(npad, dtype=jnp.int32) % N_TRASH)
    src = jnp.concatenate([edge_index[0], pad_src]).reshape(NW, steps, ch)
    dst = jnp.concatenate([edge_index[1], pad_dst]).reshape(NW, steps, ch)
    return src, dst


def kernel(x, edge_index, W1l, b1, W1r, W2l, b2, W2r):
    src2, dst2 = _pad_edges(edge_index, STEPS, CH)
    src2 = src2.reshape(NW * 2, HALF, CH)
    dst2 = dst2.reshape(NW * 2, HALF, CH)
    src1, dst1 = src2, dst2
    zf = jnp.zeros((N_ACC, DIN), jnp.float32)
    ones = jnp.ones((CH,), jnp.float32)

    aggp1, degp = _seg_sum_deg(x, src1, dst1, zf, ones)
    hl, hr = _layer1(aggp1, degp, x, W1l.T, W1r.T, W2l.T, W2r.T,
                     b1.reshape(1, DH), b2.reshape(1, DOUT))
    aggp2 = _seg_sum(hl, src2, dst2, zf)
    out, lsm = _final(aggp2, degp, hr)
    return (out, lsm)


# speedup vs baseline: 1.0697x; 1.0635x over previous
import functools

import jax
import jax.numpy as jnp
from jax import lax
from jax.experimental import pallas as pl
from jax.experimental.pallas import tpu as pltpu
from jax.experimental.pallas import tpu_sc as plsc

N = 10000
E = 320000
DIN = 128
DH = 256
DOUT = 128

NC = 2
NS = 16
NW = NC * NS
CH = 128
STAGES = 3
HALF = 28
STEPS = STAGES * HALF
N_TRASH = 8
N_ACC = N + N_TRASH

RPT = 624
R_TAIL_BASE = 16 * RPT
R_TAIL = N_ACC - 16 * RPT
W_TAIL = N - 16 * RPT
NP = 10240


def _row_ranges(sid):
    return sid * RPT


def _seg_sum_body(with_deg, *refs):
    if with_deg:
        (feat_hbm, src_hbm, dst_hbm, zf_hbm, ones_hbm,
         aggp_hbm, degp_hbm,
         src_v, dst_v, rows_a, rows_b, ones_v, dbuf, agg_sh, deg_sh,
         sem) = refs
    else:
        (feat_hbm, src_hbm, dst_hbm, zf_hbm,
         aggp_hbm,
         src_v, dst_v, rows_a, rows_b, agg_sh, sem) = refs

    cid = lax.axis_index("c")
    sid = lax.axis_index("s")
    wid = sid * NC + cid
    r0 = sid * RPT

    pltpu.sync_copy(zf_hbm.at[pl.ds(r0, RPT)], agg_sh.at[pl.ds(r0, RPT)])
    if with_deg:
        @pl.loop(0, RPT // 16)
        def _(i):
            dbuf[pl.ds(i * 16, 16)] = jnp.zeros((16,), jnp.float32)
        pltpu.sync_copy(dbuf, deg_sh.at[pl.ds(r0, RPT)])

    @pl.when(sid == NS - 1)
    def _():
        pltpu.sync_copy(zf_hbm.at[pl.ds(R_TAIL_BASE, R_TAIL)],
                        agg_sh.at[pl.ds(R_TAIL_BASE, R_TAIL)])
        if with_deg:
            pltpu.sync_copy(dbuf.at[pl.ds(0, R_TAIL)],
                            deg_sh.at[pl.ds(R_TAIL_BASE, R_TAIL)])

    if with_deg:
        pltpu.sync_copy(ones_hbm, ones_v)

    plsc.subcore_barrier()

    def issue(s, buf):
        return pltpu.async_copy(feat_hbm.at[src_v.at[s]], buf, sem)

    def scatter(s, buf):
        pltpu.sync_copy(buf, agg_sh.at[dst_v.at[s]], add=True)
        if with_deg:
            pltpu.sync_copy(ones_v, deg_sh.at[dst_v.at[s]], add=True)

    for h in range(STAGES):
        pltpu.sync_copy(src_hbm.at[STAGES * wid + h], src_v)
        pltpu.sync_copy(dst_hbm.at[STAGES * wid + h], dst_v)
        issue(0, rows_a)

        @pl.loop(0, HALF // 2)
        def _(t):
            s0 = 2 * t
            s1 = s0 + 1
            pltpu.make_async_copy(feat_hbm.at[pl.ds(0, CH)],
                                  rows_a, sem).wait()
            db = issue(s1, rows_b)
            scatter(s0, rows_a)
            db.wait()

            @pl.when(s1 + 1 < HALF)
            def _():
                issue(s1 + 1, rows_a)
            scatter(s1, rows_b)

    plsc.subcore_barrier()

    pltpu.sync_copy(agg_sh.at[pl.ds(r0, RPT)],
                    aggp_hbm.at[pl.ds(cid * NP + r0, RPT)])
    if with_deg:
        pltpu.sync_copy(deg_sh.at[pl.ds(r0, RPT)], dbuf)
        pltpu.sync_copy(dbuf, degp_hbm.at[pl.ds(cid * NP + r0, RPT)])

    @pl.when(sid == NS - 1)
    def _():
        pltpu.sync_copy(agg_sh.at[pl.ds(R_TAIL_BASE, W_TAIL)],
                        aggp_hbm.at[pl.ds(cid * NP + R_TAIL_BASE, W_TAIL)])
        if with_deg:
            pltpu.sync_copy(deg_sh.at[pl.ds(R_TAIL_BASE, W_TAIL)],
                            dbuf.at[pl.ds(0, W_TAIL)])
            pltpu.sync_copy(dbuf.at[pl.ds(0, W_TAIL)],
                            degp_hbm.at[pl.ds(cid * NP + R_TAIL_BASE, W_TAIL)])


def _make_seg_sum(with_deg):
    mesh = plsc.VectorSubcoreMesh(core_axis_name="c", subcore_axis_name="s",
                                  num_cores=NC, num_subcores=NS)
    if with_deg:
        out_type = (
            jax.ShapeDtypeStruct((NC * NP, DIN), jnp.float32),
            jax.ShapeDtypeStruct((NC * NP,), jnp.float32),
        )
    else:
        out_type = jax.ShapeDtypeStruct((NC * NP, DIN), jnp.float32)
    scratch = [
        pltpu.VMEM((HALF, CH), jnp.int32),
        pltpu.VMEM((HALF, CH), jnp.int32),
        pltpu.VMEM((CH, DIN), jnp.float32),
        pltpu.VMEM((CH, DIN), jnp.float32),
    ]
    if with_deg:
        scratch.append(pltpu.VMEM((CH,), jnp.float32))
        scratch.append(pltpu.VMEM((RPT,), jnp.float32))
    scratch.append(pltpu.VMEM_SHARED((N_ACC, DIN), jnp.float32))
    if with_deg:
        scratch.append(pltpu.VMEM_SHARED((N_ACC,), jnp.float32))
    scratch.append(pltpu.SemaphoreType.DMA)
    return pl.kernel(
        functools.partial(_seg_sum_body, with_deg),
        out_type=out_type,
        mesh=mesh,
        scratch_types=scratch,
    )


@functools.cache
def _get_seg_sum(with_deg):
    return _make_seg_sum(with_deg)


def _seg_sum_deg(*args):
    return _get_seg_sum(True)(*args)


def _seg_sum(*args):
    return _get_seg_sum(False)(*args)


RB = 1024
GRID = NP // RB


def _layer1_body(a0, a1, d0, d1, x, w1l, w1r, w2l, w2r, b1, b2, hl_o, hr_o):
    agg = a0[...] + a1[...]
    deg = jnp.maximum(d0[...] + d1[...], 1.0)
    mean = agg * (1.0 / deg)[:, None]
    h = mean @ w1l[...] + x[...] @ w1r[...] + b1[...]
    h = jnp.maximum(h, 0.0)
    hl_o[...] = h @ w2l[...]
    hr_o[...] = h @ w2r[...] + b2[...]


def _layer1(aggp, degp, x, w1l, w1r, w2l, w2r, b1, b2):
    return pl.pallas_call(
        _layer1_body,
        grid=(GRID,),
        in_specs=[
            pl.BlockSpec((RB, DIN), lambda i: (i, 0)),
            pl.BlockSpec((RB, DIN), lambda i: (GRID + i, 0)),
            pl.BlockSpec((RB,), lambda i: (i,)),
            pl.BlockSpec((RB,), lambda i: (GRID + i,)),
            pl.BlockSpec((RB, DIN), lambda i: (i, 0)),
            pl.BlockSpec((DIN, DH), lambda i: (0, 0)),
            pl.BlockSpec((DIN, DH), lambda i: (0, 0)),
            pl.BlockSpec((DH, DOUT), lambda i: (0, 0)),
            pl.BlockSpec((DH, DOUT), lambda i: (0, 0)),
            pl.BlockSpec((1, DH), lambda i: (0, 0)),
            pl.BlockSpec((1, DOUT), lambda i: (0, 0)),
        ],
        out_specs=[
            pl.BlockSpec((RB, DOUT), lambda i: (i, 0)),
            pl.BlockSpec((RB, DOUT), lambda i: (i, 0)),
        ],
        out_shape=[
            jax.ShapeDtypeStruct((N, DOUT), jnp.float32),
            jax.ShapeDtypeStruct((N, DOUT), jnp.float32),
        ],
    )(aggp, aggp, degp, degp, x, w1l, w1r, w2l, w2r, b1, b2)


def _final_body(a0, a1, d0, d1, hr, out_o, lsm_o):
    agg = a0[...] + a1[...]
    deg = jnp.maximum(d0[...] + d1[...], 1.0)
    o = agg * (1.0 / deg)[:, None] + hr[...]
    m = jnp.max(o, axis=1, keepdims=True)
    e = jnp.exp(o - m)
    s = jnp.sum(e, axis=1, keepdims=True)
    out_o[...] = o
    lsm_o[...] = (o - m) - jnp.log(s)


def _final(aggp2, degp, hr):
    return pl.pallas_call(
        _final_body,
        grid=(GRID,),
        in_specs=[
            pl.BlockSpec((RB, DOUT), lambda i: (i, 0)),
            pl.BlockSpec((RB, DOUT), lambda i: (GRID + i, 0)),
            pl.BlockSpec((RB,), lambda i: (i,)),
            pl.BlockSpec((RB,), lambda i: (GRID + i,)),
            pl.BlockSpec((RB, DOUT), lambda i: (i, 0)),
        ],
        out_specs=[
            pl.BlockSpec((RB, DOUT), lambda i: (i, 0)),
            pl.BlockSpec((RB, DOUT), lambda i: (i, 0)),
        ],
        out_shape=[
            jax.ShapeDtypeStruct((N, DOUT), jnp.float32),
            jax.ShapeDtypeStruct((N, DOUT), jnp.float32),
        ],
    )(aggp2, aggp2, degp, degp, hr)


def _pad_edges(edge_index, steps, ch):
    npad = NW * steps * ch - E
    pad_src = (jnp.arange(npad, dtype=jnp.int32) * 37) % N
    pad_dst = N + (jnp.arange(npad, dtype=jnp.int32) % N_TRASH)
    src = jnp.concatenate([edge_index[0], pad_src]).reshape(NW, steps, ch)
    dst = jnp.concatenate([edge_index[1], pad_dst]).reshape(NW, steps, ch)
    return src, dst


def kernel(x, edge_index, W1l, b1, W1r, W2l, b2, W2r):
    src2, dst2 = _pad_edges(edge_index, STEPS, CH)
    src2 = src2.reshape(NW * STAGES, HALF, CH)
    dst2 = dst2.reshape(NW * STAGES, HALF, CH)
    src1, dst1 = src2, dst2
    zf = jnp.zeros((N_ACC, DIN), jnp.float32)
    ones = jnp.ones((CH,), jnp.float32)

    aggp1, degp = _seg_sum_deg(x, src1, dst1, zf, ones)
    hl, hr = _layer1(aggp1, degp, x, W1l.T, W1r.T, W2l.T, W2r.T,
                     b1.reshape(1, DH), b2.reshape(1, DOUT))
    aggp2 = _seg_sum(hl, src2, dst2, zf)
    out, lsm = _final(aggp2, degp, hr)
    return (out, lsm)
